# Initial kernel scaffold; baseline (speedup 1.0000x reference)
import jax
import jax.numpy as jnp
from jax.experimental import pallas as pl


def kernel(x, pre_encode_b, W, WT, b1, b2):
    raise NotImplementedError("write your pallas kernel here")


# speedup vs baseline: 12.6407x; 12.6407x over previous
import functools

import numpy as np
import jax
import jax.numpy as jnp
from jax.experimental import pallas as pl
from jax.experimental.pallas import tpu as pltpu

_K = 32


def _fused_body(x_ref, peb_ref, w_ref, wt_ref, b1_ref, b2_ref, o_ref, *, k):
    xc = x_ref[...] - peb_ref[...]
    pre = jnp.dot(xc, wt_ref[...], preferred_element_type=jnp.float32)
    pre = pre + b1_ref[...]

    b = jax.lax.bitcast_convert_type(pre, jnp.int32)
    su = b ^ ((b >> jnp.int32(31)) & jnp.int32(0x7FFFFFFF))
    minint = jnp.int32(np.int32(-2**31))

    rows = pre.shape[0]
    p = jnp.zeros((rows, 1), jnp.int32)
    for i in range(31, -1, -1):
        bit = jnp.int32(np.uint32(1 << i).astype(np.int32))
        c = p | bit
        sc = c ^ minint
        cnt = jnp.sum((su >= sc).astype(jnp.int32), axis=1, keepdims=True)
        p = jnp.where(cnt >= k, c, p)
    sp = p ^ minint
    masked = jnp.where(su >= sp, pre, jnp.float32(0.0))

    out = jnp.dot(masked, w_ref[...], preferred_element_type=jnp.float32)
    o_ref[...] = out + b2_ref[...]


def kernel(x, pre_encode_b, W, WT, b1, b2):
    tokens, input_size = x.shape
    hidden = WT.shape[1]
    bt = 256
    grid = (tokens // bt,)
    out = pl.pallas_call(
        functools.partial(_fused_body, k=_K),
        grid=grid,
        in_specs=[
            pl.BlockSpec((bt, input_size), lambda i: (i, 0)),
            pl.BlockSpec((1, hidden), lambda i: (0, 0)),
            pl.BlockSpec((hidden, input_size), lambda i: (0, 0)),
            pl.BlockSpec((input_size, hidden), lambda i: (0, 0)),
            pl.BlockSpec((1, hidden), lambda i: (0, 0)),
            pl.BlockSpec((1, input_size), lambda i: (0, 0)),
        ],
        out_specs=pl.BlockSpec((bt, input_size), lambda i: (i, 0)),
        out_shape=jax.ShapeDtypeStruct((tokens, input_size), jnp.float32),
        compiler_params=pltpu.CompilerParams(
            dimension_semantics=("parallel",),
        ),
    )(
        x,
        pre_encode_b.reshape(1, hidden),
        W,
        WT,
        b1.reshape(1, hidden),
        b2.reshape(1, input_size),
    )
    return out
